# Initial kernel scaffold; baseline (speedup 1.0000x reference)
import jax
import jax.numpy as jnp
from jax.experimental import pallas as pl


def kernel(pointcloud, params):
    raise NotImplementedError("write your pallas kernel here")


# speedup vs baseline: 6.6475x; 6.6475x over previous
import jax
import jax.numpy as jnp
from jax.experimental import pallas as pl

_NPOINTS = [512, 256, 128, 64]
_RADIUS = [[0.01, 0.02], [0.02, 0.04], [0.04, 0.08], [0.08, 0.16]]
_NSAMPLE = [[16, 32], [16, 32], [16, 32], [16, 32]]

_HI = jax.lax.Precision.DEFAULT
_EX = jax.lax.Precision.HIGHEST


def _cumsum_last(x):
    n = x.shape[-1]
    k = 1
    while k < n:
        shifted = jnp.concatenate(
            [jnp.zeros_like(x[..., :k]), x[..., : n - k]], axis=-1
        )
        x = x + shifted
        k *= 2
    return x


def _fps_call(xt, npoint):
    b, _, n = xt.shape

    def kern(xyz_ref, new_ref):
        x = xyz_ref[...]
        iota = jax.lax.broadcasted_iota(jnp.int32, (b, n), 1)
        iop = jax.lax.broadcasted_iota(jnp.int32, (1, npoint), 1)

        def body(i, carry):
            dists, far, acc = carry
            oh = (iota == far).astype(jnp.float32)
            cent = jnp.sum(oh[:, None, :] * x, axis=2)
            slot = (iop == i).astype(jnp.float32)
            acc = acc + cent[:, :, None] * slot[None, :, :]
            d = jnp.sum((x - cent[:, :, None]) ** 2, axis=1)
            dists = jnp.minimum(dists, d)
            far = jnp.argmax(dists, axis=1).astype(jnp.int32)[:, None]
            return dists, far, acc

        _, _, acc = jax.lax.fori_loop(
            0,
            npoint,
            body,
            (
                jnp.full((b, n), 1e10, jnp.float32),
                jnp.zeros((b, 1), jnp.int32),
                jnp.zeros((b, 3, npoint), jnp.float32),
            ),
        )
        new_ref[...] = acc

    return pl.pallas_call(
        kern,
        out_shape=jax.ShapeDtypeStruct((b, 3, npoint), jnp.float32),
    )(xt)


def _group_mlp_call(xt, ft, new_xt, radius, nsample, layers, c_chunk):
    b, _, n = xt.shape
    npoint = new_xt.shape[1]
    cin = ft.shape[1]
    dout = layers[-1][0].shape[1]
    r2 = float(radius) * float(radius)
    cc = c_chunk
    p = cc * nsample

    w_args = []
    for (w, g, be) in layers:
        w_args += [w, g.reshape(-1, 1), be.reshape(-1, 1)]

    def kern(xt_ref, ft_ref, new_ref, *rest):
        out_ref = rest[-1]
        wr = rest[:-1]
        x = xt_ref[0]
        f = ft_ref[0]
        nx = new_ref[0]

        a2 = jnp.sum(nx * nx, axis=1)[:, None]
        b2 = jnp.sum(x * x, axis=0)[None, :]
        prod = jax.lax.dot_general(
            nx, x, (((1,), (0,)), ((), ())), precision=_HI
        )
        sq = a2 + b2 - 2.0 * prod
        maskf = (sq < r2).astype(jnp.float32)
        csum = _cumsum_last(maskf)
        rank = csum - maskf
        cnt = csum[:, n - 1:n]

        srange = jax.lax.broadcasted_iota(
            jnp.int32, (1, nsample, 1), 1
        ).astype(jnp.float32)
        base = maskf[:, None, :] * (rank[:, None, :] == srange).astype(
            jnp.float32
        )
        first = maskf * (rank == 0.0).astype(jnp.float32)
        e0 = (
            jax.lax.broadcasted_iota(jnp.int32, (1, n), 1) == 0
        ).astype(jnp.float32)
        fb = jnp.where(cnt > 0.0, first, e0)
        need_pad = srange >= cnt[:, :, None]
        sel = jnp.where(need_pad, fb[:, None, :], base).reshape(p, n)

        pts = jnp.concatenate([x, f], axis=0)
        g = jax.lax.dot_general(
            pts, sel, (((1,), (1,)), ((), ())), precision=_EX
        )

        rowi = jax.lax.broadcasted_iota(jnp.int32, (cc, p), 0)
        coli = jax.lax.broadcasted_iota(jnp.int32, (cc, p), 1)
        rep = (coli // nsample == rowi).astype(jnp.float32)
        nxrep = jax.lax.dot_general(
            nx, rep, (((0,), (0,)), ((), ())), precision=_EX
        )
        h = jnp.concatenate([g[:3] - nxrep, g[3:]], axis=0)

        for li in range(len(wr) // 3):
            w = wr[3 * li][...]
            ga = wr[3 * li + 1][...]
            be = wr[3 * li + 2][...]
            h = jax.lax.dot_general(
                w, h, (((0,), (0,)), ((), ())), precision=_HI
            )
            h = h * ga + be
            h = jnp.maximum(h, 0.0)

        k = 1
        while k < nsample:
            shifted = jnp.concatenate(
                [h[:, k:], jnp.zeros((h.shape[0], k), jnp.float32)], axis=1
            )
            h = jnp.maximum(h, shifted)
            k *= 2

        rowp = jax.lax.broadcasted_iota(jnp.int32, (p, cc), 0)
        colc = jax.lax.broadcasted_iota(jnp.int32, (p, cc), 1)
        selmat = (rowp == colc * nsample).astype(jnp.float32)
        out = jax.lax.dot_general(
            selmat, h, (((0,), (1,)), ((), ())), precision=_EX
        )
        out_ref[0] = out

    grid = (b, npoint // cc)
    in_specs = [
        pl.BlockSpec((1, 3, n), lambda bi, ci: (bi, 0, 0)),
        pl.BlockSpec((1, cin, n), lambda bi, ci: (bi, 0, 0)),
        pl.BlockSpec((1, cc, 3), lambda bi, ci: (bi, ci, 0)),
    ]
    for wa in w_args:
        in_specs.append(pl.BlockSpec(wa.shape, lambda bi, ci: (0, 0)))
    return pl.pallas_call(
        kern,
        grid=grid,
        in_specs=in_specs,
        out_specs=pl.BlockSpec((1, cc, dout), lambda bi, ci: (bi, ci, 0)),
        out_shape=jax.ShapeDtypeStruct((b, npoint, dout), jnp.float32),
    )(xt, ft, new_xt, *w_args)


def _fp_call(ut, kt, ufeats, kfeats, layers):
    b, _, nu = ut.shape
    nk = kt.shape[2]
    ckf = kfeats.shape[2]
    cu = ufeats.shape[2]
    dout = layers[-1][0].shape[1]

    w_args = []
    for (w, g, be) in layers:
        w_args += [w, g.reshape(1, -1), be.reshape(1, -1)]

    def kern(u_ref, k_ref, uf_ref, kf_ref, *rest):
        out_ref = rest[-1]
        wr = rest[:-1]
        u = u_ref[0]
        kx = k_ref[0]
        uf = uf_ref[0]
        kf = kf_ref[0]

        a2 = jnp.sum(u * u, axis=0)[:, None]
        b2 = jnp.sum(kx * kx, axis=0)[None, :]
        prod = jax.lax.dot_general(
            u, kx, (((0,), (0,)), ((), ())), precision=_HI
        )
        sq = a2 + b2 - 2.0 * prod

        iok = jax.lax.broadcasted_iota(jnp.int32, (1, nk), 1)
        m = jnp.zeros((nu, nk), jnp.float32)
        rsum = jnp.zeros((nu, 1), jnp.float32)
        work = sq
        for _ in range(3):
            d = jnp.min(work, axis=1, keepdims=True)
            i = jnp.argmin(work, axis=1)[:, None]
            selv = (iok == i)
            r = 1.0 / (d + 1e-8)
            m = m + r * selv.astype(jnp.float32)
            rsum = rsum + r
            work = jnp.where(selv, 1e30, work)
        m = m / rsum

        interp = jax.lax.dot_general(
            m, kf, (((1,), (0,)), ((), ())), precision=_EX
        )
        h = jnp.concatenate([interp, uf], axis=1)
        for li in range(len(wr) // 3):
            w = wr[3 * li][...]
            ga = wr[3 * li + 1][...]
            be = wr[3 * li + 2][...]
            h = jax.lax.dot_general(
                h, w, (((1,), (0,)), ((), ())), precision=_HI
            )
            h = h * ga + be
            h = jnp.maximum(h, 0.0)
        out_ref[0] = h

    in_specs = [
        pl.BlockSpec((1, 3, nu), lambda bi: (bi, 0, 0)),
        pl.BlockSpec((1, 3, nk), lambda bi: (bi, 0, 0)),
        pl.BlockSpec((1, nu, cu), lambda bi: (bi, 0, 0)),
        pl.BlockSpec((1, nk, ckf), lambda bi: (bi, 0, 0)),
    ]
    for wa in w_args:
        in_specs.append(pl.BlockSpec(wa.shape, lambda bi: (0, 0)))
    return pl.pallas_call(
        kern,
        grid=(b,),
        in_specs=in_specs,
        out_specs=pl.BlockSpec((1, nu, dout), lambda bi: (bi, 0, 0)),
        out_shape=jax.ShapeDtypeStruct((b, nu, dout), jnp.float32),
    )(ut, kt, ufeats, kfeats, *w_args)


_C_CHUNK = [16, 128, 128, 64]


def kernel(pointcloud, params):
    xt = jnp.transpose(pointcloud[..., 0:3], (0, 2, 1))
    feats = pointcloud[..., 3:]
    l_xt = [xt]
    l_feats = [feats]
    for k in range(len(_NPOINTS)):
        new_xt = _fps_call(l_xt[k], _NPOINTS[k])
        new_r = jnp.transpose(new_xt, (0, 2, 1))
        ft = jnp.transpose(l_feats[k], (0, 2, 1))
        outs = []
        for radius, nsample, layers in zip(
            _RADIUS[k], _NSAMPLE[k], params["sa"][k]
        ):
            outs.append(
                _group_mlp_call(
                    l_xt[k], ft, new_r, radius, nsample, layers,
                    _C_CHUNK[k],
                )
            )
        l_xt.append(new_xt)
        l_feats.append(jnp.concatenate(outs, axis=-1))
    nfp = len(params["fp"])
    for i in range(-1, -(nfp + 1), -1):
        l_feats[i - 1] = _fp_call(
            l_xt[i - 1], l_xt[i], l_feats[i - 1], l_feats[i],
            params["fp"][i],
        )
    return jnp.transpose(l_feats[0], (0, 2, 1))
